# baseline (speedup 1.0000x reference)
import jax
import jax.numpy as jnp
from jax import lax
from jax.experimental import pallas as pl
from jax.experimental.pallas import tpu as pltpu
from jax.experimental.pallas import tpu_sc as plsc

B, S, D = 4, 2048, 1024
NC, NS = 2, 16
NW = NC * NS
SEG = S // NW
SUB = 16
QPS = SEG // SUB
NT = B * QPS
GPR = D // 16


def _body(x_ref, tok_ref, pos_ref, out_ref, idx_v, pbuf, buf0, buf1,
          psem, gs0, gs1, ws0, ws1):
    c = lax.axis_index("c")
    s = lax.axis_index("s")
    wid = s * NC + c
    bufs = (buf0, buf1)
    gsems = (gs0, gs1)
    wsems = (ws0, ws1)

    pltpu.sync_copy(x_ref.at[wid], idx_v)

    def start_gather(t):
        return pltpu.async_copy(tok_ref.at[idx_v.at[t]], bufs[t % 2],
                                gsems[t % 2])

    gd = [start_gather(0), None]
    pd = pltpu.async_copy(pos_ref.at[pl.ds(wid * SEG, SEG)], pbuf, psem)
    wb = [None, None]
    for t in range(NT):
        p = t % 2
        if t + 1 < NT:
            if wb[1 - p] is not None:
                wb[1 - p].wait()
                wb[1 - p] = None
            gd[1 - p] = start_gather(t + 1)
        gd[p].wait()
        if t == 0:
            pd.wait()

        q = t % QPS
        cur = bufs[p]

        @plsc.parallel_loop(0, SUB * GPR, unroll=4)
        def _add(i):
            r = i // GPR
            k = (i % GPR) * 16
            plsc.addupdate(cur.at[r, pl.ds(k, 16)],
                           pbuf[q * SUB + r, pl.ds(k, 16)])

        b = t // QPS
        base = b * S + wid * SEG + q * SUB
        wb[p] = pltpu.async_copy(cur, out_ref.at[pl.ds(base, SUB)], wsems[p])
    for d in wb:
        if d is not None:
            d.wait()


@jax.jit
def _emb(xr, token_table, pos_table):
    kern = pl.kernel(
        _body,
        out_type=jax.ShapeDtypeStruct((B * S, D), jnp.float32),
        mesh=plsc.VectorSubcoreMesh(core_axis_name="c", subcore_axis_name="s"),
        scratch_types=[
            pltpu.VMEM((NT, SUB), jnp.int32),
            pltpu.VMEM((SEG, D), jnp.float32),
            pltpu.VMEM((SUB, D), jnp.float32),
            pltpu.VMEM((SUB, D), jnp.float32),
            pltpu.SemaphoreType.DMA,
            pltpu.SemaphoreType.DMA,
            pltpu.SemaphoreType.DMA,
            pltpu.SemaphoreType.DMA,
            pltpu.SemaphoreType.DMA,
        ],
    )
    return kern(xr, token_table, pos_table)


def kernel(x, token_table, pos_table):
    xr = (x.astype(jnp.int32)
          .reshape(B, NW, NT // B, SUB)
          .transpose(1, 0, 2, 3)
          .reshape(NW, NT, SUB))
    out = _emb(xr, token_table, pos_table)
    return out.reshape(B, S, D)


# speedup vs baseline: 1.1502x; 1.1502x over previous
import jax
import jax.numpy as jnp
from jax import lax
from jax.experimental import pallas as pl
from jax.experimental.pallas import tpu as pltpu
from jax.experimental.pallas import tpu_sc as plsc

B, S, D = 4, 2048, 1024
NC, NS = 2, 16
NW = NC * NS
SEG = S // NW
SUB = 8
QN = SEG // SUB
GR = B * SUB
GPR = D // 16
NBUF = 3


def _body(x_ref, tok_ref, pos_ref, out_ref, idx_v, pb0, pb1, pb2,
          cb0, cb1, cb2, ps0, ps1, ps2, gs0, gs1, gs2, ws0, ws1, ws2):
    c = lax.axis_index("c")
    s = lax.axis_index("s")
    wid = s * NC + c
    pbufs = (pb0, pb1, pb2)
    combos = (cb0, cb1, cb2)
    psems = (ps0, ps1, ps2)
    gsems = (gs0, gs1, gs2)
    wsems = (ws0, ws1, ws2)

    pltpu.sync_copy(x_ref.at[wid], idx_v)

    def start_fetch(j):
        p = j % NBUF
        g = pltpu.async_copy(tok_ref.at[idx_v.at[j]], combos[p], gsems[p])
        d = pltpu.async_copy(pos_ref.at[pl.ds(wid * SEG + j * SUB, SUB)],
                             pbufs[p], psems[p])
        return (g, d)

    fd = [start_fetch(0), start_fetch(1), None]
    wb = [None, None, None]
    for j in range(QN):
        p = j % NBUF
        if j + 2 < QN:
            p2 = (j + 2) % NBUF
            if wb[p2] is not None:
                for d in wb[p2]:
                    d.wait()
                wb[p2] = None
            fd[p2] = start_fetch(j + 2)
        fd[p][0].wait()
        fd[p][1].wait()

        cur = combos[p]
        pb = pbufs[p]

        @plsc.parallel_loop(0, SUB * GPR, unroll=4)
        def _add(i):
            r = i // GPR
            k = (i % GPR) * 16
            v = pb[r, pl.ds(k, 16)]
            for b in range(B):
                plsc.addupdate(cur.at[b * SUB + r, pl.ds(k, 16)], v)

        base = wid * SEG + j * SUB
        wb[p] = [
            pltpu.async_copy(cur.at[pl.ds(b * SUB, SUB)],
                             out_ref.at[pl.ds(b * S + base, SUB)], wsems[p])
            for b in range(B)
        ]
    for ds_ in wb:
        if ds_ is not None:
            for d in ds_:
                d.wait()


@jax.jit
def _emb(xr, token_table, pos_table):
    kern = pl.kernel(
        _body,
        out_type=jax.ShapeDtypeStruct((B * S, D), jnp.float32),
        mesh=plsc.VectorSubcoreMesh(core_axis_name="c", subcore_axis_name="s"),
        scratch_types=[
            pltpu.VMEM((QN, GR), jnp.int32),
            pltpu.VMEM((SUB, D), jnp.float32),
            pltpu.VMEM((SUB, D), jnp.float32),
            pltpu.VMEM((SUB, D), jnp.float32),
            pltpu.VMEM((GR, D), jnp.float32),
            pltpu.VMEM((GR, D), jnp.float32),
            pltpu.VMEM((GR, D), jnp.float32),
        ] + [pltpu.SemaphoreType.DMA] * 9,
    )
    return kern(xr, token_table, pos_table)


def kernel(x, token_table, pos_table):
    xr = (x.astype(jnp.int32)
          .reshape(B, NW, QN, SUB)
          .transpose(1, 2, 0, 3)
          .reshape(NW, QN, GR))
    out = _emb(xr, token_table, pos_table)
    return out.reshape(B, S, D)
